# baseline (speedup 1.0000x reference)
import functools

import jax
import jax.numpy as jnp
from jax import lax
from jax.experimental import pallas as pl
from jax.experimental.pallas import tpu as pltpu
from jax.experimental.pallas import tpu_sc as plsc

EMB_K = 16
CHUNK = 128


@functools.lru_cache(maxsize=None)
def _build(batch, n_rows):
    info = plsc.get_sparse_core_info()
    nw = info.num_cores * info.num_subcores
    bpw = batch // nw
    n_chunks = bpw // CHUNK
    n_vregs = bpw // 16
    mesh = plsc.VectorSubcoreMesh(core_axis_name="c", subcore_axis_name="s")

    @functools.partial(
        pl.kernel,
        mesh=mesh,
        compiler_params=pltpu.CompilerParams(
            use_tc_tiling_on_sc=False, needs_layout_passes=False),
        out_type=jax.ShapeDtypeStruct((batch,), jnp.float32),
        scratch_types=[
            pltpu.VMEM((n_chunks, CHUNK), jnp.int32),
            pltpu.VMEM((n_chunks, CHUNK), jnp.int32),
            pltpu.VMEM((bpw,), jnp.float32),
            pltpu.VMEM((bpw,), jnp.float32),
            pltpu.VMEM((bpw,), jnp.float32),
            pltpu.SemaphoreType.DMA,
        ],
    )
    def k(uidx_hbm, vidx_hbm, wt_hbm, ht_hbm, out_hbm,
          uidx_v, vidx_v, wbuf, hbuf, outv, sem):
        wid = lax.axis_index("s") * info.num_cores + lax.axis_index("c")
        base = wid * bpw
        for c in range(n_chunks):
            pltpu.sync_copy(uidx_hbm.at[pl.ds(base + c * CHUNK, CHUNK)],
                            uidx_v.at[c])
            pltpu.sync_copy(vidx_hbm.at[pl.ds(base + c * CHUNK, CHUNK)],
                            vidx_v.at[c])
        zero = jnp.zeros((16,), jnp.float32)
        for j in range(n_vregs):
            outv[pl.ds(j * 16, 16)] = zero

        def feature(kk, carry):
            copies = []
            for c in range(n_chunks):
                copies.append(pltpu.async_copy(
                    wt_hbm.at[kk].at[uidx_v.at[c]],
                    wbuf.at[pl.ds(c * CHUNK, CHUNK)], sem))
                copies.append(pltpu.async_copy(
                    ht_hbm.at[kk].at[vidx_v.at[c]],
                    hbuf.at[pl.ds(c * CHUNK, CHUNK)], sem))
            for cp in copies:
                cp.wait()
            for j in range(n_vregs):
                sl = pl.ds(j * 16, 16)
                outv[sl] = outv[sl] + wbuf[sl] * hbuf[sl]
            return carry

        lax.fori_loop(0, EMB_K, feature, 0)
        pltpu.sync_copy(outv, out_hbm.at[pl.ds(base, bpw)])

    return k


def kernel(x, W, H):
    uidx = x[:, 0].astype(jnp.int32)
    vidx = x[:, 1].astype(jnp.int32)
    k = _build(x.shape[0], W.shape[0])
    return k(uidx, vidx, W.T, H.T)


# speedup vs baseline: 19.4598x; 19.4598x over previous
import functools

import jax
import jax.numpy as jnp
from jax import lax
from jax.experimental import pallas as pl
from jax.experimental.pallas import tpu as pltpu
from jax.experimental.pallas import tpu_sc as plsc

EMB_K = 16
HALF = 8

_GATHER_DNUMS = lax.GatherDimensionNumbers(
    offset_dims=(), collapsed_slice_dims=(0,), start_index_map=(0,))


def _shuffle(v, perm_2d):
    return lax.gather(v, perm_2d, _GATHER_DNUMS, (1,),
                      mode=lax.GatherScatterMode.PROMISE_IN_BOUNDS)


@functools.lru_cache(maxsize=None)
def _build(batch, n_rows):
    info = plsc.get_sparse_core_info()
    nw = info.num_cores * info.num_subcores
    bpw = batch // nw
    n_pairs = bpw // 16
    mesh = plsc.VectorSubcoreMesh(core_axis_name="c", subcore_axis_name="s")

    @functools.partial(
        pl.kernel,
        mesh=mesh,
        compiler_params=pltpu.CompilerParams(needs_layout_passes=False),
        out_type=jax.ShapeDtypeStruct((batch,), jnp.float32),
        scratch_types=[
            pltpu.VMEM((bpw,), jnp.int32),
            pltpu.VMEM((bpw,), jnp.int32),
            pltpu.VMEM((2, EMB_K, HALF * 128), jnp.float32),
            pltpu.VMEM((2, EMB_K, HALF * 128), jnp.float32),
            pltpu.VMEM((bpw,), jnp.float32),
            pltpu.SemaphoreType.DMA,
            pltpu.SemaphoreType.DMA,
        ],
    )
    def k(uidx_hbm, vidx_hbm, wt_hbm, ht_hbm, out_hbm,
          uidx_v, vidx_v, wslots, hslots, outv, sem0, sem1):
        wid = lax.axis_index("s") * info.num_cores + lax.axis_index("c")
        base = wid * bpw
        pltpu.sync_copy(uidx_hbm.at[pl.ds(base, bpw)], uidx_v)
        pltpu.sync_copy(vidx_hbm.at[pl.ds(base, bpw)], vidx_v)

        lane = lax.iota(jnp.int32, 16)
        perms = [(lane ^ d).reshape(16, 1) for d in (8, 4, 2, 1)]
        sems = (sem0, sem1)

        def load_idx(i):
            sl = pl.ds(i * 16, 16)
            return uidx_v[sl], vidx_v[sl]

        def fire(uvec, vvec, half, ph):
            sem = sems[ph]
            for r in range(HALF):
                u = uvec[half * HALF + r]
                v = vvec[half * HALF + r]
                u0 = pl.multiple_of((u >> 7) << 7, 128)
                v0 = pl.multiple_of((v >> 7) << 7, 128)
                pltpu.async_copy(
                    wt_hbm.at[:, pl.ds(u0, 128)],
                    wslots.at[ph, :, pl.ds(r * 128, 128)], sem)
                pltpu.async_copy(
                    ht_hbm.at[:, pl.ds(v0, 128)],
                    hslots.at[ph, :, pl.ds(r * 128, 128)], sem)

        def drain(ph):
            sem = sems[ph]
            for r in range(HALF):
                pltpu.make_async_copy(
                    wt_hbm.at[:, pl.ds(0, 128)],
                    wslots.at[ph, :, pl.ds(r * 128, 128)], sem).wait()
                pltpu.make_async_copy(
                    ht_hbm.at[:, pl.ds(0, 128)],
                    hslots.at[ph, :, pl.ds(r * 128, 128)], sem).wait()

        def compute(uvec, vvec, half, ph, acc):
            for r in range(HALF):
                u = uvec[half * HALF + r]
                v = vvec[half * HALF + r]
                ucol = jnp.full((16,), r * 128, jnp.int32) + (u & 127)
                vcol = jnp.full((16,), r * 128, jnp.int32) + (v & 127)
                wu = plsc.load_gather(wslots.at[ph], [lane, ucol])
                hv = plsc.load_gather(hslots.at[ph], [lane, vcol])
                p = wu * hv
                for perm in perms:
                    p = p + _shuffle(p, perm)
                acc = jnp.where(lane == half * HALF + r, p, acc)
            return acc

        u0vec, v0vec = load_idx(0)
        fire(u0vec, v0vec, 0, 0)
        fire(u0vec, v0vec, 1, 1)

        def pair(i, carry):
            uvec, vvec = load_idx(i)
            nxt_u, nxt_v = load_idx(jnp.minimum(i + 1, n_pairs - 1))
            last = i >= n_pairs - 1

            drain(0)
            acc = compute(uvec, vvec, 0, 0, jnp.zeros((16,), jnp.float32))

            @pl.when(jnp.logical_not(last))
            def _():
                fire(nxt_u, nxt_v, 0, 0)

            drain(1)
            acc = compute(uvec, vvec, 1, 1, acc)

            @pl.when(jnp.logical_not(last))
            def _():
                fire(nxt_u, nxt_v, 1, 1)

            outv[pl.ds(i * 16, 16)] = acc
            return carry

        lax.fori_loop(0, n_pairs, pair, 0)
        pltpu.sync_copy(outv, out_hbm.at[pl.ds(base, bpw)])

    return k


def kernel(x, W, H):
    uidx = x[:, 0].astype(jnp.int32)
    vidx = x[:, 1].astype(jnp.int32)
    k = _build(x.shape[0], W.shape[0])
    return k(uidx, vidx, W.T, H.T)
